# Initial kernel scaffold; baseline (speedup 1.0000x reference)
import jax
import jax.numpy as jnp
from jax.experimental import pallas as pl


def kernel(output, target):
    raise NotImplementedError("write your pallas kernel here")


# speedup vs baseline: 16.7471x; 16.7471x over previous
import functools

import jax
import jax.numpy as jnp
from jax import lax
from jax.experimental import pallas as pl
from jax.experimental.pallas import tpu as pltpu
from jax.experimental.pallas import tpu_sc as plsc

N = 131072
C = 19
K = 1024
NCORES = 2
NSUB = 16
NW = NCORES * NSUB
RPT = N // NW
CHUNK = 256
NCHUNK = RPT // CHUNK
CK = C * K
TBL = 4 * CK
EPC = CHUNK * C
MAGIC = 55189
SHIFT = 20


def _sc_hist_body(out_hbm, tgt_hbm, hist_hbm, in_v, tgt_v, err_v, tbl_v):
    cid = lax.axis_index("c")
    sid = lax.axis_index("s")
    wid = sid * NCORES + cid
    iota = lax.iota(jnp.int32, 16)
    zeros16 = jnp.zeros((16,), jnp.float32)
    ones16 = jnp.ones((16,), jnp.float32)

    def zero_body(i, carry):
        tbl_v[pl.ds(i * 16, 16)] = zeros16
        return carry

    lax.fori_loop(0, TBL // 16, zero_body, 0, unroll=8)

    def chunk_body(ci, carry):
        row0 = wid * RPT + ci * CHUNK
        pltpu.sync_copy(out_hbm.at[pl.ds(row0 * C, EPC)], in_v)
        pltpu.sync_copy(tgt_hbm.at[pl.ds(row0, CHUNK)], tgt_v)

        def grp_body(gi, c2):
            r = gi * 16
            rows = r + iota
            base = rows * C
            vals = [plsc.load_gather(in_v, [base + c]) for c in range(C)]
            exps = [jnp.exp(v) for v in vals]
            s = exps[0]
            for c in range(1, C):
                s = s + exps[c]
            rcp = 1.0 / s
            tv = tgt_v[pl.ds(r, 16)]
            for c in range(C):
                p = exps[c] * rcp
                fg = tv == c
                err = jnp.where(fg, 1.0 - p, p)
                plsc.store_scatter(err_v, [base + c], err)
            return c2

        lax.fori_loop(0, CHUNK // 16, grp_body, 0)

        def scat_body(vi, c3):
            jb = vi * 16
            j = jb + iota
            row = lax.shift_right_logical(j * MAGIC, SHIFT)
            cc = j - row * C
            tr = plsc.load_gather(tgt_v, [row])
            fg = cc == tr
            e = err_v[pl.ds(jb, 16)]
            b = jnp.minimum((e * float(K)).astype(jnp.int32), K - 1)
            g = cc * K + b
            plsc.addupdate_scatter(tbl_v, [g], ones16)
            plsc.addupdate_scatter(tbl_v, [g + CK], e)
            plsc.addupdate_scatter(tbl_v, [g + 2 * CK], ones16, mask=fg)
            plsc.addupdate_scatter(tbl_v, [g + 3 * CK], e, mask=fg)
            return c3

        lax.fori_loop(0, EPC // 16, scat_body, 0, unroll=4)
        return carry

    lax.fori_loop(0, NCHUNK, chunk_body, 0)
    pltpu.sync_copy(tbl_v, hist_hbm.at[wid])


def _finish_body(hist_ref, out_ref):
    tot = jnp.sum(hist_ref[...], axis=0)
    bi = lax.broadcasted_iota(jnp.int32, (K, K), 0)
    bj = lax.broadcasted_iota(jnp.int32, (K, K), 1)
    m = (bi > bj).astype(jnp.float32)
    cum = jax.lax.dot_general(
        tot, m, (((1,), (0,)), ((), ())),
        preferred_element_type=jnp.float32)
    cntA = tot[0:C]
    sumA = tot[C:2 * C]
    cntP = tot[2 * C:3 * C]
    sumP = tot[3 * C:4 * C]
    cumA = cum[0:C]
    cumP = cum[2 * C:3 * C]
    cumB = cumA - cumP
    P = jnp.sum(cntP, axis=1, keepdims=True)
    denom = jnp.maximum(P + cumB, 0.5)
    lo = lax.broadcasted_iota(jnp.int32, (C, K), 1).astype(jnp.float32) \
        * (1.0 / K)
    j0w = cumA / denom * (1.0 / K)
    cntN = cntA - cntP
    sumN = sumA - sumP
    corr = (sumP - cntP * lo) / denom \
        + (sumN - cntN * lo) * (P - cumP) / (denom * denom)
    loss_c = jnp.sum(j0w + corr, axis=1)
    present = (P[:, 0] > 0).astype(jnp.float32)
    loss = jnp.sum(loss_c * present) / jnp.maximum(jnp.sum(present), 1.0)
    out_ref[...] = jnp.reshape(loss, (1, 1))


_sc_hist = pl.kernel(
    _sc_hist_body,
    out_type=jax.ShapeDtypeStruct((NW, TBL), jnp.float32),
    mesh=plsc.VectorSubcoreMesh(
        core_axis_name="c", subcore_axis_name="s",
        num_cores=NCORES, num_subcores=NSUB),
    compiler_params=pltpu.CompilerParams(needs_layout_passes=False),
    scratch_types=[
        pltpu.VMEM((EPC,), jnp.float32),
        pltpu.VMEM((CHUNK,), jnp.int32),
        pltpu.VMEM((EPC,), jnp.float32),
        pltpu.VMEM((TBL,), jnp.float32),
    ],
)

_finish = pl.pallas_call(
    _finish_body,
    out_shape=jax.ShapeDtypeStruct((1, 1), jnp.float32),
)


@jax.jit
def kernel(output, target):
    hist = _sc_hist(output.reshape(-1), target)
    loss = _finish(hist.reshape(NW, 4 * C, K))
    return loss.reshape(())
